# Initial kernel scaffold; baseline (speedup 1.0000x reference)
import jax
import jax.numpy as jnp
from jax.experimental import pallas as pl


def kernel(style_id, embed_weight):
    raise NotImplementedError("write your pallas kernel here")


# speedup vs baseline: 1.9184x; 1.9184x over previous
import functools

import jax
import jax.numpy as jnp
from jax import lax
from jax.experimental import pallas as pl
from jax.experimental.pallas import tpu as pltpu
from jax.experimental.pallas import tpu_sc as plsc

_NUM_STYLES = 1000
_STYLE_DIM = 64
_BATCH = 16384

_NC = 2
_NS = 16
_NW = _NC * _NS
_B_PER_W = _BATCH // _NW

_mesh = plsc.VectorSubcoreMesh(core_axis_name="c", subcore_axis_name="s")


@functools.partial(
    pl.kernel,
    mesh=_mesh,
    out_type=jax.ShapeDtypeStruct((_BATCH, _STYLE_DIM), jnp.float32),
    scratch_types=[
        pltpu.VMEM((_B_PER_W,), jnp.int32),
        pltpu.VMEM((_B_PER_W, _STYLE_DIM), jnp.float32),
        pltpu.SemaphoreType.DMA,
    ],
    compiler_params=pltpu.CompilerParams(use_tc_tiling_on_sc=False),
)
def _gather_kernel(table_hbm, idx_hbm, out_hbm, idx_v, rows_v, sem):
    wid = lax.axis_index("s") * _NC + lax.axis_index("c")
    base = wid * _B_PER_W
    pltpu.sync_copy(idx_hbm.at[pl.ds(base, _B_PER_W)], idx_v)
    pltpu.async_copy(table_hbm.at[idx_v], rows_v, sem).wait()
    pltpu.sync_copy(rows_v, out_hbm.at[pl.ds(base, _B_PER_W)])


def kernel(style_id, embed_weight):
    return _gather_kernel(embed_weight, style_id.astype(jnp.int32))


# speedup vs baseline: 2.1276x; 1.1090x over previous
import functools

import jax
import jax.numpy as jnp
from jax import lax
from jax.experimental import pallas as pl
from jax.experimental.pallas import tpu as pltpu
from jax.experimental.pallas import tpu_sc as plsc

_NUM_STYLES = 1000
_STYLE_DIM = 64
_PAD_DIM = 128
_BATCH = 16384

_NC = 2
_NS = 16
_NW = _NC * _NS
_B_PER_W = _BATCH // _NW

_mesh = plsc.VectorSubcoreMesh(core_axis_name="c", subcore_axis_name="s")


@functools.partial(
    pl.kernel,
    mesh=_mesh,
    out_type=jax.ShapeDtypeStruct((_BATCH, _PAD_DIM), jnp.float32),
    scratch_types=[
        pltpu.VMEM((_B_PER_W,), jnp.int32),
        pltpu.VMEM((_B_PER_W, _PAD_DIM), jnp.float32),
        pltpu.SemaphoreType.DMA,
    ],
)
def _gather_kernel(table_hbm, idx_hbm, out_hbm, idx_v, rows_v, sem):
    wid = lax.axis_index("s") * _NC + lax.axis_index("c")
    base = wid * _B_PER_W
    pltpu.sync_copy(idx_hbm.at[pl.ds(base, _B_PER_W)], idx_v)
    pltpu.async_copy(table_hbm.at[idx_v], rows_v, sem).wait()
    pltpu.sync_copy(rows_v, out_hbm.at[pl.ds(base, _B_PER_W)])


def kernel(style_id, embed_weight):
    table128 = jnp.pad(embed_weight, ((0, 0), (0, _PAD_DIM - _STYLE_DIM)))
    out128 = _gather_kernel(table128, style_id.astype(jnp.int32))
    return out128[:, :_STYLE_DIM]
